# baseline (speedup 1.0000x reference)
import jax
import jax.numpy as jnp
from jax.experimental import pallas as pl


def kernel(x, edge_index, batch, W1, b1, W2, b2, Ws, bs):
    N = x.shape[0]
    G = 64
    static_x = jnp.ones_like(x)
    src = edge_index[0]
    dst = edge_index[1]
    agg = jax.ops.segment_sum(static_x[src], dst, num_segments=N)
    h = jax.nn.relu(agg @ W1 + b1)
    agg = jax.ops.segment_sum(h[src], dst, num_segments=N)
    h = jax.nn.relu(agg @ W2 + b2)
    pooled = jax.ops.segment_sum(h, batch, num_segments=G)
    counts = jax.ops.segment_sum(jnp.ones((N, 1), jnp.float32), batch, num_segments=G)
    state = pooled / jnp.maximum(counts, 1.0)
    return jax.nn.sigmoid(state @ Ws + bs)


# speedup vs baseline: 20.9948x; 20.9948x over previous
import functools

import jax
import jax.numpy as jnp
from jax import lax
from jax.experimental import pallas as pl
from jax.experimental.pallas import tpu as pltpu
from jax.experimental.pallas import tpu_sc as plsc

N = 10000
E = 320000
D = 128
G = 64
K = 128
OVD = K - 2
CH = 2000

NC, NS = 2, 16
EPC = E // NC
EPT2 = EPC // NS
EPT1 = E // NS
NCH1 = EPT1 // CH
NCH2 = EPT2 // CH
ZB = 16000
CSLICE = (N * K) // NS


def _sc_body(src_hbm, dst_hbm, out_hbm, deg_sh, c_sh,
             idx_v, dst_v, val_v, fidx_v, ovidx_v, ones_v, zbuf_v):
    cid = lax.axis_index("c")
    sid = lax.axis_index("s")

    def fill_zero(i, _):
        zbuf_v[pl.ds(i * 16, 16)] = jnp.zeros((16,), jnp.float32)
        return 0
    lax.fori_loop(0, ZB // 16, fill_zero, 0)

    def fill_one(i, _):
        ones_v[pl.ds(i * 16, 16)] = jnp.ones((16,), jnp.float32)
        return 0
    lax.fori_loop(0, CH // 16, fill_one, 0)

    for z in range(CSLICE // ZB):
        pltpu.sync_copy(zbuf_v, c_sh.at[pl.ds(sid * CSLICE + z * ZB, ZB)])

    @pl.when(sid == 0)
    def _():
        pltpu.sync_copy(zbuf_v.at[pl.ds(0, N)], deg_sh)

    plsc.subcore_barrier()

    def p1(j, _):
        base = sid * EPT1 + j * CH
        pltpu.sync_copy(dst_hbm.at[pl.ds(base, CH)], dst_v)
        pltpu.sync_copy(ones_v, deg_sh.at[dst_v], add=True)
        return 0
    lax.fori_loop(0, NCH1, p1, 0)

    plsc.subcore_barrier()

    def p2(j, _):
        base = cid * EPC + sid * EPT2 + j * CH
        pltpu.sync_copy(src_hbm.at[pl.ds(base, CH)], idx_v)
        pltpu.sync_copy(dst_hbm.at[pl.ds(base, CH)], dst_v)
        pltpu.sync_copy(deg_sh.at[idx_v], val_v)

        def vec(i, _):
            d = val_v[pl.ds(i * 16, 16)]
            di = d.astype(jnp.int32)
            dstw = dst_v[pl.ds(i * 16, 16)]
            dc = jnp.minimum(di, OVD)
            fidx_v[pl.ds(i * 16, 16)] = dstw * K + dc
            ovidx_v[pl.ds(i * 16, 16)] = dstw * K + (K - 1)
            val_v[pl.ds(i * 16, 16)] = jnp.where(di >= OVD, d, 0.0)
            return 0
        lax.fori_loop(0, CH // 16, vec, 0)

        pltpu.sync_copy(ones_v, c_sh.at[fidx_v], add=True)
        pltpu.sync_copy(val_v, c_sh.at[ovidx_v], add=True)
        return 0
    lax.fori_loop(0, NCH2, p2, 0)

    plsc.subcore_barrier()

    pltpu.sync_copy(c_sh.at[pl.ds(sid * CSLICE, CSLICE)],
                    out_hbm.at[cid, pl.ds(sid * CSLICE, CSLICE)])


_sc_build = pl.kernel(
    _sc_body,
    out_type=jax.ShapeDtypeStruct((NC, N * K), jnp.float32),
    mesh=plsc.VectorSubcoreMesh(core_axis_name="c", subcore_axis_name="s"),
    scratch_types=[
        pltpu.VMEM_SHARED((N,), jnp.float32),
        pltpu.VMEM_SHARED((N * K,), jnp.float32),
        pltpu.VMEM((CH,), jnp.int32),
        pltpu.VMEM((CH,), jnp.int32),
        pltpu.VMEM((CH,), jnp.float32),
        pltpu.VMEM((CH,), jnp.int32),
        pltpu.VMEM((CH,), jnp.int32),
        pltpu.VMEM((CH,), jnp.float32),
        pltpu.VMEM((ZB,), jnp.float32),
    ],
)


def _tc_body(cp_ref, bt_ref, w1_ref, b1_ref, w2_ref, b2_ref, ws_ref, bs_ref,
             out_ref):
    hi = lax.Precision.HIGHEST
    C = cp_ref[0] + cp_ref[1]
    c1 = jnp.sum(w1_ref[...], axis=0, keepdims=True)
    b1 = b1_ref[...]
    gp = jnp.where(c1 > 0, c1, 0.0)
    g0 = jnp.where(c1 > 0, b1, jnp.where(c1 == 0, jnp.maximum(b1, 0.0), 0.0))
    rowi = lax.broadcasted_iota(jnp.int32, (K, D), 0)
    drow = rowi.astype(jnp.float32)
    F = jnp.maximum(drow * c1 + b1, 0.0)
    F = jnp.where(rowi == OVD, g0, jnp.where(rowi == K - 1, gp, F))
    agg2 = lax.dot_general(C, F, (((1,), (0,)), ((), ())), precision=hi)
    h2 = jnp.maximum(
        lax.dot_general(agg2, w2_ref[...], (((1,), (0,)), ((), ())),
                        precision=hi) + b2_ref[...], 0.0)
    bt = bt_ref[...]
    oh = (lax.broadcasted_iota(jnp.int32, (G, N), 0) == bt).astype(jnp.float32)
    pooled = lax.dot_general(oh, h2, (((1,), (0,)), ((), ())), precision=hi)
    counts = jnp.sum(oh, axis=1, keepdims=True)
    state = pooled / jnp.maximum(counts, 1.0)
    z = lax.dot_general(state, ws_ref[...], (((1,), (0,)), ((), ())),
                        precision=hi) + bs_ref[...]
    out_ref[...] = 1.0 / (1.0 + jnp.exp(-z))


_tc_dense = pl.pallas_call(
    _tc_body,
    out_shape=jax.ShapeDtypeStruct((G, 1), jnp.float32),
)


def kernel(x, edge_index, batch, W1, b1, W2, b2, Ws, bs):
    cp = _sc_build(edge_index[0], edge_index[1]).reshape(NC, N, K)
    return _tc_dense(cp, batch.reshape(1, N), W1, b1.reshape(1, D),
                     W2, b2.reshape(1, D), Ws, bs.reshape(1, 1))


# speedup vs baseline: 26.1972x; 1.2478x over previous
import jax
import jax.numpy as jnp
from jax import lax
from jax.experimental import pallas as pl
from jax.experimental.pallas import tpu as pltpu
from jax.experimental.pallas import tpu_sc as plsc

N = 10000
E = 320000
D = 128
G = 64
K = 128
OVD = K - 2

NC, NS = 2, 16
EPT1 = E // NS
EPT2 = E // (NC * NS)
CH1 = 2000
CH2 = 2000
ZB = 16000
CSLICE = (N * K) // NS


def _sc_body(src_hbm, dst_hbm, out_hbm, deg_sh, c_sh,
             idx_c, dst_c, dval_c, fidx_c, ones_v, zbuf_v):
    cid = lax.axis_index("c")
    sid = lax.axis_index("s")
    ones16 = jnp.ones((16,), jnp.float32)
    zeros16 = jnp.zeros((16,), jnp.float32)

    def fill_zero(i, _):
        zbuf_v[pl.ds(i * 16, 16)] = zeros16
        return 0
    lax.fori_loop(0, ZB // 16, fill_zero, 0)

    def fill_one(i, _):
        ones_v[pl.ds(i * 16, 16)] = ones16
        return 0
    lax.fori_loop(0, CH1 // 16, fill_one, 0)

    for z in range(CSLICE // ZB):
        pltpu.sync_copy(zbuf_v, c_sh.at[pl.ds(sid * CSLICE + z * ZB, ZB)])

    @pl.when(sid == 0)
    def _():
        pltpu.sync_copy(zbuf_v.at[pl.ds(0, N)], deg_sh)

    plsc.subcore_barrier()

    for j in range(EPT1 // CH1):
        pltpu.sync_copy(dst_hbm.at[pl.ds(sid * EPT1 + j * CH1, CH1)], dst_c)
        pltpu.sync_copy(ones_v.at[pl.ds(0, CH1)], deg_sh.at[dst_c], add=True)

    plsc.subcore_barrier()

    base = cid * (E // NC) + sid * EPT2
    mxv = zeros16
    for j in range(EPT2 // CH2):
        pltpu.sync_copy(src_hbm.at[pl.ds(base + j * CH2, CH2)], idx_c)
        pltpu.sync_copy(dst_hbm.at[pl.ds(base + j * CH2, CH2)],
                        dst_c.at[pl.ds(0, CH2)])
        pltpu.sync_copy(deg_sh.at[idx_c], dval_c)

        def vec(i, mx):
            d = dval_c[pl.ds(i * 16, 16)]
            di = d.astype(jnp.int32)
            dstw = dst_c[pl.ds(i * 16, 16)]
            dc = jnp.minimum(di, OVD)
            fidx_c[pl.ds(i * 16, 16)] = dstw * K + dc
            return jnp.maximum(mx, d)
        mxv = lax.fori_loop(0, CH2 // 16, vec, mxv)
        pltpu.sync_copy(ones_v.at[pl.ds(0, CH2)], c_sh.at[fidx_c], add=True)

    mx = mxv[0]
    for t in range(1, 16):
        mx = jnp.maximum(mx, mxv[t])

    @pl.when(mx >= float(OVD))
    def _():
        for j in range(EPT2 // CH2):
            pltpu.sync_copy(src_hbm.at[pl.ds(base + j * CH2, CH2)], idx_c)
            pltpu.sync_copy(dst_hbm.at[pl.ds(base + j * CH2, CH2)],
                            dst_c.at[pl.ds(0, CH2)])
            pltpu.sync_copy(deg_sh.at[idx_c], dval_c)

            def ovv(i, _):
                d = dval_c[pl.ds(i * 16, 16)]
                di = d.astype(jnp.int32)
                dstw = dst_c[pl.ds(i * 16, 16)]
                fidx_c[pl.ds(i * 16, 16)] = dstw * K + (K - 1)
                dval_c[pl.ds(i * 16, 16)] = jnp.where(di >= OVD, d, 0.0)
                return 0
            lax.fori_loop(0, CH2 // 16, ovv, 0)
            pltpu.sync_copy(dval_c, c_sh.at[fidx_c], add=True)

    plsc.subcore_barrier()

    pltpu.sync_copy(c_sh.at[pl.ds(sid * CSLICE, CSLICE)],
                    out_hbm.at[cid, pl.ds(sid * CSLICE, CSLICE)])


_sc_build = pl.kernel(
    _sc_body,
    out_type=jax.ShapeDtypeStruct((NC, N * K), jnp.float32),
    mesh=plsc.VectorSubcoreMesh(core_axis_name="c", subcore_axis_name="s"),
    scratch_types=[
        pltpu.VMEM_SHARED((N,), jnp.float32),
        pltpu.VMEM_SHARED((N * K,), jnp.float32),
        pltpu.VMEM((CH2,), jnp.int32),
        pltpu.VMEM((CH1,), jnp.int32),
        pltpu.VMEM((CH2,), jnp.float32),
        pltpu.VMEM((CH2,), jnp.int32),
        pltpu.VMEM((CH1,), jnp.float32),
        pltpu.VMEM((ZB,), jnp.float32),
    ],
)


def _tc_body(cp_ref, bt_ref, w1_ref, b1_ref, w2_ref, b2_ref, ws_ref, bs_ref,
             out_ref):
    hi = lax.Precision.HIGHEST
    C = cp_ref[0] + cp_ref[1]
    c1 = jnp.sum(w1_ref[...], axis=0, keepdims=True)
    b1 = b1_ref[...]
    gp = jnp.where(c1 > 0, c1, 0.0)
    g0 = jnp.where(c1 > 0, b1, jnp.where(c1 == 0, jnp.maximum(b1, 0.0), 0.0))
    rowi = lax.broadcasted_iota(jnp.int32, (K, D), 0)
    drow = rowi.astype(jnp.float32)
    F = jnp.maximum(drow * c1 + b1, 0.0)
    F = jnp.where(rowi == OVD, g0, jnp.where(rowi == K - 1, gp, F))
    agg2 = lax.dot_general(C, F, (((1,), (0,)), ((), ())), precision=hi)
    h2 = jnp.maximum(
        lax.dot_general(agg2, w2_ref[...], (((1,), (0,)), ((), ())),
                        precision=hi) + b2_ref[...], 0.0)
    bt = bt_ref[...]
    oh = (lax.broadcasted_iota(jnp.int32, (G, N), 0) == bt).astype(jnp.float32)
    pooled = lax.dot_general(oh, h2, (((1,), (0,)), ((), ())), precision=hi)
    counts = jnp.sum(oh, axis=1, keepdims=True)
    state = pooled / jnp.maximum(counts, 1.0)
    z = lax.dot_general(state, ws_ref[...], (((1,), (0,)), ((), ())),
                        precision=hi) + bs_ref[...]
    out_ref[...] = 1.0 / (1.0 + jnp.exp(-z))


_tc_dense = pl.pallas_call(
    _tc_body,
    out_shape=jax.ShapeDtypeStruct((G, 1), jnp.float32),
)


def kernel(x, edge_index, batch, W1, b1, W2, b2, Ws, bs):
    cp = _sc_build(edge_index[0], edge_index[1]).reshape(NC, N, K)
    return _tc_dense(cp, batch.reshape(1, N), W1, b1.reshape(1, D),
                     W2, b2.reshape(1, D), Ws, bs.reshape(1, 1))


# speedup vs baseline: 33.3598x; 1.2734x over previous
import jax
import jax.numpy as jnp
from jax import lax
from jax.experimental import pallas as pl
from jax.experimental.pallas import tpu as pltpu
from jax.experimental.pallas import tpu_sc as plsc

N = 10000
E = 320000
D = 128
G = 64
K = 128
OVD = K - 2

NC, NS = 2, 16
EPT1 = E // NS
EPT2 = E // (NC * NS)
CH1 = 2000
CH2 = 2000
ZB = 16000
CSLICE = (N * K) // NS


def _sc_body(e_hbm, out_hbm, deg_sh, c_sh,
             idx_c, dst_c, dval_c, fidx_c, ones_v, zbuf_v):
    cid = lax.axis_index("c")
    sid = lax.axis_index("s")
    ones16 = jnp.ones((16,), jnp.float32)
    zeros16 = jnp.zeros((16,), jnp.float32)

    def fill_zero(i, _):
        zbuf_v[pl.ds(i * 16, 16)] = zeros16
        return 0
    lax.fori_loop(0, ZB // 16, fill_zero, 0)

    def fill_one(i, _):
        ones_v[pl.ds(i * 16, 16)] = ones16
        return 0
    lax.fori_loop(0, CH1 // 16, fill_one, 0)

    for z in range(CSLICE // ZB):
        pltpu.sync_copy(zbuf_v, c_sh.at[pl.ds(sid * CSLICE + z * ZB, ZB)])

    @pl.when(sid == 0)
    def _():
        pltpu.sync_copy(zbuf_v.at[pl.ds(0, N)], deg_sh)

    plsc.subcore_barrier()

    for j in range(EPT1 // CH1):
        pltpu.sync_copy(e_hbm.at[pl.ds(E + sid * EPT1 + j * CH1, CH1)], dst_c)
        pltpu.sync_copy(ones_v.at[pl.ds(0, CH1)], deg_sh.at[dst_c], add=True)

    plsc.subcore_barrier()

    base = cid * (E // NC) + sid * EPT2
    mxv = zeros16
    for j in range(EPT2 // CH2):
        pltpu.sync_copy(e_hbm.at[pl.ds(base + j * CH2, CH2)], idx_c)
        pltpu.sync_copy(e_hbm.at[pl.ds(E + base + j * CH2, CH2)],
                        dst_c.at[pl.ds(0, CH2)])
        pltpu.sync_copy(deg_sh.at[idx_c], dval_c)

        def vec(i, mx):
            d = dval_c[pl.ds(i * 16, 16)]
            di = d.astype(jnp.int32)
            dstw = dst_c[pl.ds(i * 16, 16)]
            dc = jnp.minimum(di, OVD)
            fidx_c[pl.ds(i * 16, 16)] = dstw * K + dc
            return jnp.maximum(mx, d)
        mxv = lax.fori_loop(0, CH2 // 16, vec, mxv)
        pltpu.sync_copy(ones_v.at[pl.ds(0, CH2)], c_sh.at[fidx_c], add=True)

    mx = mxv[0]
    for t in range(1, 16):
        mx = jnp.maximum(mx, mxv[t])

    @pl.when(mx >= float(OVD))
    def _():
        for j in range(EPT2 // CH2):
            pltpu.sync_copy(e_hbm.at[pl.ds(base + j * CH2, CH2)], idx_c)
            pltpu.sync_copy(e_hbm.at[pl.ds(E + base + j * CH2, CH2)],
                            dst_c.at[pl.ds(0, CH2)])
            pltpu.sync_copy(deg_sh.at[idx_c], dval_c)

            def ovv(i, _):
                d = dval_c[pl.ds(i * 16, 16)]
                di = d.astype(jnp.int32)
                dstw = dst_c[pl.ds(i * 16, 16)]
                fidx_c[pl.ds(i * 16, 16)] = dstw * K + (K - 1)
                dval_c[pl.ds(i * 16, 16)] = jnp.where(di >= OVD, d, 0.0)
                return 0
            lax.fori_loop(0, CH2 // 16, ovv, 0)
            pltpu.sync_copy(dval_c, c_sh.at[fidx_c], add=True)

    plsc.subcore_barrier()

    pltpu.sync_copy(c_sh.at[pl.ds(sid * CSLICE, CSLICE)],
                    out_hbm.at[cid, pl.ds(sid * CSLICE, CSLICE)])


_sc_build = pl.kernel(
    _sc_body,
    out_type=jax.ShapeDtypeStruct((NC, N * K), jnp.float32),
    mesh=plsc.VectorSubcoreMesh(core_axis_name="c", subcore_axis_name="s"),
    scratch_types=[
        pltpu.VMEM_SHARED((N,), jnp.float32),
        pltpu.VMEM_SHARED((N * K,), jnp.float32),
        pltpu.VMEM((CH2,), jnp.int32),
        pltpu.VMEM((CH1,), jnp.int32),
        pltpu.VMEM((CH2,), jnp.float32),
        pltpu.VMEM((CH2,), jnp.int32),
        pltpu.VMEM((CH1,), jnp.float32),
        pltpu.VMEM((ZB,), jnp.float32),
    ],
)


def _tc_body(cp_ref, bt_ref, w1_ref, b1_ref, w2_ref, b2_ref, ws_ref, bs_ref,
             out_ref):
    mm = lambda a, b: lax.dot_general(a, b, (((1,), (0,)), ((), ())))
    C = cp_ref[0] + cp_ref[1]
    c1 = jnp.sum(w1_ref[...], axis=0, keepdims=True)
    b1 = b1_ref[...]
    gp = jnp.where(c1 > 0, c1, 0.0)
    g0 = jnp.where(c1 > 0, b1, jnp.where(c1 == 0, jnp.maximum(b1, 0.0), 0.0))
    rowi = lax.broadcasted_iota(jnp.int32, (K, D), 0)
    drow = rowi.astype(jnp.float32)
    F = jnp.maximum(drow * c1 + b1, 0.0)
    F = jnp.where(rowi == OVD, g0, jnp.where(rowi == K - 1, gp, F))
    M = mm(F, w2_ref[...])
    h2 = jnp.maximum(mm(C, M) + b2_ref[...], 0.0)
    v = mm(h2, ws_ref[...])
    bt = bt_ref[...]
    oh = (lax.broadcasted_iota(jnp.int32, (G, N), 0) == bt).astype(jnp.float32)
    counts = jnp.sum(oh, axis=1, keepdims=True)
    sv = mm(oh, v)
    z = sv / jnp.maximum(counts, 1.0) + bs_ref[...]
    out_ref[...] = 1.0 / (1.0 + jnp.exp(-z))


_tc_dense = pl.pallas_call(
    _tc_body,
    out_shape=jax.ShapeDtypeStruct((G, 1), jnp.float32),
)


def kernel(x, edge_index, batch, W1, b1, W2, b2, Ws, bs):
    cp = _sc_build(edge_index.reshape(2 * E)).reshape(NC, N, K)
    return _tc_dense(cp, batch.reshape(1, N), W1, b1.reshape(1, D),
                     W2, b2.reshape(1, D), Ws, bs.reshape(1, 1))


# speedup vs baseline: 37.0284x; 1.1100x over previous
import jax
import jax.numpy as jnp
from jax import lax
from jax.experimental import pallas as pl
from jax.experimental.pallas import tpu as pltpu
from jax.experimental.pallas import tpu_sc as plsc

N = 10000
E = 320000
D = 128
G = 64
K = 128
OVD = K - 2

NC, NS = 2, 16
EPT1 = E // NS
EPT2 = E // (NC * NS)
CH1 = 2000
CH2 = 2000
ZB = 8000
CSLICE = (N * K) // NS


def _sc_body(e_hbm, out_hbm, deg_sh, c_sh,
             idx_a, idx_b, dst1_a, dst1_b, dst2_a, dst2_b,
             dval_a, dval_b, fidx_a, fidx_b, ones_v, zbuf_v,
             sg_a, sg_b, ss_a, ss_b):
    cid = lax.axis_index("c")
    sid = lax.axis_index("s")
    ones16 = jnp.ones((16,), jnp.float32)
    zeros16 = jnp.zeros((16,), jnp.float32)
    idxs = [idx_a, idx_b]
    dst1s = [dst1_a, dst1_b]
    dst2s = [dst2_a, dst2_b]
    dvals = [dval_a, dval_b]
    fidxs = [fidx_a, fidx_b]
    sgs = [sg_a, sg_b]
    sss = [ss_a, ss_b]

    def fill_zero(i, _):
        zbuf_v[pl.ds(i * 16, 16)] = zeros16
        return 0
    lax.fori_loop(0, ZB // 16, fill_zero, 0)

    def fill_one(i, _):
        ones_v[pl.ds(i * 16, 16)] = ones16
        return 0
    lax.fori_loop(0, CH2 // 16, fill_one, 0)

    for z in range(CSLICE // ZB):
        pltpu.sync_copy(zbuf_v, c_sh.at[pl.ds(sid * CSLICE + z * ZB, ZB)])

    @pl.when(sid == 0)
    def _():
        pltpu.sync_copy(zbuf_v, deg_sh.at[pl.ds(0, ZB)])
        pltpu.sync_copy(zbuf_v.at[pl.ds(0, N - ZB)],
                        deg_sh.at[pl.ds(ZB, N - ZB)])

    plsc.subcore_barrier()

    hs = [None, None]
    for j in range(EPT1 // CH1):
        b = j & 1
        if hs[b] is not None:
            hs[b].wait()
        pltpu.sync_copy(e_hbm.at[pl.ds(E + sid * EPT1 + j * CH1, CH1)],
                        dst1s[b])
        hs[b] = pltpu.async_copy(ones_v, deg_sh.at[dst1s[b]], sss[b],
                                 add=True)
    for h in hs:
        h.wait()

    plsc.subcore_barrier()

    base = cid * (E // NC) + sid * EPT2
    NJ = EPT2 // CH2
    pltpu.sync_copy(e_hbm.at[pl.ds(base, CH2)], idxs[0])
    pltpu.sync_copy(e_hbm.at[pl.ds(E + base, CH2)], dst2s[0])
    hg = [None, None]
    hg[0] = pltpu.async_copy(deg_sh.at[idxs[0]], dvals[0], sgs[0])
    hsc = [None, None]
    mxv = zeros16
    for j in range(NJ):
        b = j & 1
        if hsc[b] is not None:
            hsc[b].wait()
        hg[b].wait()
        if j + 1 < NJ:
            nb = (j + 1) & 1
            pltpu.sync_copy(e_hbm.at[pl.ds(base + (j + 1) * CH2, CH2)],
                            idxs[nb])
            pltpu.sync_copy(e_hbm.at[pl.ds(E + base + (j + 1) * CH2, CH2)],
                            dst2s[nb])
            hg[nb] = pltpu.async_copy(deg_sh.at[idxs[nb]], dvals[nb], sgs[nb])

        dval_c = dvals[b]
        dst_c = dst2s[b]
        fidx_c = fidxs[b]

        def vec(i, mx):
            d = dval_c[pl.ds(i * 16, 16)]
            di = d.astype(jnp.int32)
            dstw = dst_c[pl.ds(i * 16, 16)]
            dc = jnp.minimum(di, OVD)
            fidx_c[pl.ds(i * 16, 16)] = dstw * K + dc
            return jnp.maximum(mx, d)
        mxv = lax.fori_loop(0, CH2 // 16, vec, mxv)
        hsc[b] = pltpu.async_copy(ones_v, c_sh.at[fidx_c], sss[b], add=True)
    for h in hsc:
        if h is not None:
            h.wait()

    mx = mxv[0]
    for t in range(1, 16):
        mx = jnp.maximum(mx, mxv[t])

    @pl.when(mx >= float(OVD))
    def _():
        for j in range(NJ):
            pltpu.sync_copy(e_hbm.at[pl.ds(base + j * CH2, CH2)], idx_a)
            pltpu.sync_copy(e_hbm.at[pl.ds(E + base + j * CH2, CH2)], dst2_a)
            pltpu.sync_copy(deg_sh.at[idx_a], dval_a)

            def ovv(i, _):
                d = dval_a[pl.ds(i * 16, 16)]
                di = d.astype(jnp.int32)
                dstw = dst2_a[pl.ds(i * 16, 16)]
                fidx_a[pl.ds(i * 16, 16)] = dstw * K + (K - 1)
                dval_a[pl.ds(i * 16, 16)] = jnp.where(di >= OVD, d, 0.0)
                return 0
            lax.fori_loop(0, CH2 // 16, ovv, 0)
            pltpu.sync_copy(dval_a, c_sh.at[fidx_a], add=True)

    plsc.subcore_barrier()

    pltpu.sync_copy(c_sh.at[pl.ds(sid * CSLICE, CSLICE)],
                    out_hbm.at[cid, pl.ds(sid * CSLICE, CSLICE)])


_sc_build = pl.kernel(
    _sc_body,
    out_type=jax.ShapeDtypeStruct((NC, N * K), jnp.float32),
    mesh=plsc.VectorSubcoreMesh(core_axis_name="c", subcore_axis_name="s"),
    scratch_types=[
        pltpu.VMEM_SHARED((N,), jnp.float32),
        pltpu.VMEM_SHARED((N * K,), jnp.float32),
        pltpu.VMEM((CH2,), jnp.int32),
        pltpu.VMEM((CH2,), jnp.int32),
        pltpu.VMEM((CH1,), jnp.int32),
        pltpu.VMEM((CH1,), jnp.int32),
        pltpu.VMEM((CH2,), jnp.int32),
        pltpu.VMEM((CH2,), jnp.int32),
        pltpu.VMEM((CH2,), jnp.float32),
        pltpu.VMEM((CH2,), jnp.float32),
        pltpu.VMEM((CH2,), jnp.int32),
        pltpu.VMEM((CH2,), jnp.int32),
        pltpu.VMEM((CH2,), jnp.float32),
        pltpu.VMEM((ZB,), jnp.float32),
        pltpu.SemaphoreType.DMA,
        pltpu.SemaphoreType.DMA,
        pltpu.SemaphoreType.DMA,
        pltpu.SemaphoreType.DMA,
    ],
)


def _tc_body(cp_ref, bt_ref, w1_ref, b1_ref, w2_ref, b2_ref, ws_ref, bs_ref,
             out_ref):
    mm = lambda a, b: lax.dot_general(a, b, (((1,), (0,)), ((), ())))
    C = cp_ref[0] + cp_ref[1]
    c1 = jnp.sum(w1_ref[...], axis=0, keepdims=True)
    b1 = b1_ref[...]
    gp = jnp.where(c1 > 0, c1, 0.0)
    g0 = jnp.where(c1 > 0, b1, jnp.where(c1 == 0, jnp.maximum(b1, 0.0), 0.0))
    rowi = lax.broadcasted_iota(jnp.int32, (K, D), 0)
    drow = rowi.astype(jnp.float32)
    F = jnp.maximum(drow * c1 + b1, 0.0)
    F = jnp.where(rowi == OVD, g0, jnp.where(rowi == K - 1, gp, F))
    M = mm(F, w2_ref[...])
    h2 = jnp.maximum(mm(C, M) + b2_ref[...], 0.0)
    v = mm(h2, ws_ref[...])
    bt = bt_ref[...]
    oh = (lax.broadcasted_iota(jnp.int32, (G, N), 0) == bt).astype(jnp.float32)
    counts = jnp.sum(oh, axis=1, keepdims=True)
    sv = mm(oh, v)
    z = sv / jnp.maximum(counts, 1.0) + bs_ref[...]
    out_ref[...] = 1.0 / (1.0 + jnp.exp(-z))


_tc_dense = pl.pallas_call(
    _tc_body,
    out_shape=jax.ShapeDtypeStruct((G, 1), jnp.float32),
)


def kernel(x, edge_index, batch, W1, b1, W2, b2, Ws, bs):
    cp = _sc_build(edge_index.reshape(2 * E)).reshape(NC, N, K)
    return _tc_dense(cp, batch.reshape(1, N), W1, b1.reshape(1, D),
                     W2, b2.reshape(1, D), Ws, bs.reshape(1, 1))


# speedup vs baseline: 42.7743x; 1.1552x over previous
import jax
import jax.numpy as jnp
from jax import lax
from jax.experimental import pallas as pl
from jax.experimental.pallas import tpu as pltpu
from jax.experimental.pallas import tpu_sc as plsc

N = 10000
E = 320000
D = 128
G = 64
K = 128
OVD = K - 2

NC, NS = 2, 16
EPT1 = E // NS
EPT2 = E // (NC * NS)
CH1 = 2000
CH2 = 2000
ZB = 8000
CSLICE = (N * K) // NS


def _sc_body(e_hbm, out0_hbm, out1_hbm, deg_sh, c_sh,
             idx_a, idx_b, dst1_a, dst1_b, dst2_a, dst2_b,
             dval_a, dval_b, fidx_a, fidx_b, ones_v, zbuf_v,
             sg_a, sg_b, ss_a, ss_b):
    cid = lax.axis_index("c")
    sid = lax.axis_index("s")
    ones16 = jnp.ones((16,), jnp.float32)
    zeros16 = jnp.zeros((16,), jnp.float32)
    idxs = [idx_a, idx_b]
    dst1s = [dst1_a, dst1_b]
    dst2s = [dst2_a, dst2_b]
    dvals = [dval_a, dval_b]
    fidxs = [fidx_a, fidx_b]
    sgs = [sg_a, sg_b]
    sss = [ss_a, ss_b]

    def fill_zero(i, _):
        zbuf_v[pl.ds(i * 16, 16)] = zeros16
        return 0
    lax.fori_loop(0, ZB // 16, fill_zero, 0)

    def fill_one(i, _):
        ones_v[pl.ds(i * 16, 16)] = ones16
        return 0
    lax.fori_loop(0, CH2 // 16, fill_one, 0)

    for z in range(CSLICE // ZB):
        pltpu.sync_copy(zbuf_v, c_sh.at[pl.ds(sid * CSLICE + z * ZB, ZB)])

    @pl.when(sid == 0)
    def _():
        pltpu.sync_copy(zbuf_v, deg_sh.at[pl.ds(0, ZB)])
        pltpu.sync_copy(zbuf_v.at[pl.ds(0, N - ZB)],
                        deg_sh.at[pl.ds(ZB, N - ZB)])

    plsc.subcore_barrier()

    hs = [None, None]
    for j in range(EPT1 // CH1):
        b = j & 1
        if hs[b] is not None:
            hs[b].wait()
        pltpu.sync_copy(e_hbm.at[pl.ds(E + sid * EPT1 + j * CH1, CH1)],
                        dst1s[b])
        hs[b] = pltpu.async_copy(ones_v, deg_sh.at[dst1s[b]], sss[b],
                                 add=True)
    for h in hs:
        h.wait()

    plsc.subcore_barrier()

    base = cid * (E // NC) + sid * EPT2
    NJ = EPT2 // CH2
    pltpu.sync_copy(e_hbm.at[pl.ds(base, CH2)], idxs[0])
    pltpu.sync_copy(e_hbm.at[pl.ds(E + base, CH2)], dst2s[0])
    hg = [None, None]
    hg[0] = pltpu.async_copy(deg_sh.at[idxs[0]], dvals[0], sgs[0])
    hsc = [None, None]
    mxv = zeros16
    for j in range(NJ):
        b = j & 1
        if hsc[b] is not None:
            hsc[b].wait()
        hg[b].wait()
        if j + 1 < NJ:
            nb = (j + 1) & 1
            pltpu.sync_copy(e_hbm.at[pl.ds(base + (j + 1) * CH2, CH2)],
                            idxs[nb])
            pltpu.sync_copy(e_hbm.at[pl.ds(E + base + (j + 1) * CH2, CH2)],
                            dst2s[nb])
            hg[nb] = pltpu.async_copy(deg_sh.at[idxs[nb]], dvals[nb], sgs[nb])

        dval_c = dvals[b]
        dst_c = dst2s[b]
        fidx_c = fidxs[b]

        def vec(i, mx):
            d = dval_c[pl.ds(i * 16, 16)]
            di = d.astype(jnp.int32)
            dstw = dst_c[pl.ds(i * 16, 16)]
            dc = jnp.minimum(di, OVD)
            fidx_c[pl.ds(i * 16, 16)] = dstw * K + dc
            return jnp.maximum(mx, d)
        mxv = lax.fori_loop(0, CH2 // 16, vec, mxv)
        hsc[b] = pltpu.async_copy(ones_v, c_sh.at[fidx_c], sss[b], add=True)
    for h in hsc:
        if h is not None:
            h.wait()

    mx = mxv[0]
    for t in range(1, 16):
        mx = jnp.maximum(mx, mxv[t])

    @pl.when(mx >= float(OVD))
    def _():
        for j in range(NJ):
            pltpu.sync_copy(e_hbm.at[pl.ds(base + j * CH2, CH2)], idx_a)
            pltpu.sync_copy(e_hbm.at[pl.ds(E + base + j * CH2, CH2)], dst2_a)
            pltpu.sync_copy(deg_sh.at[idx_a], dval_a)

            def ovv(i, _):
                d = dval_a[pl.ds(i * 16, 16)]
                di = d.astype(jnp.int32)
                dstw = dst2_a[pl.ds(i * 16, 16)]
                fidx_a[pl.ds(i * 16, 16)] = dstw * K + (K - 1)
                dval_a[pl.ds(i * 16, 16)] = jnp.where(di >= OVD, d, 0.0)
                return 0
            lax.fori_loop(0, CH2 // 16, ovv, 0)
            pltpu.sync_copy(dval_a, c_sh.at[fidx_a], add=True)

    plsc.subcore_barrier()

    @pl.when(cid == 0)
    def _():
        pltpu.sync_copy(c_sh.at[pl.ds(sid * CSLICE, CSLICE)],
                        out0_hbm.at[pl.ds(sid * CSLICE, CSLICE)])

    @pl.when(cid == 1)
    def _():
        pltpu.sync_copy(c_sh.at[pl.ds(sid * CSLICE, CSLICE)],
                        out1_hbm.at[pl.ds(sid * CSLICE, CSLICE)])


_sc_build = pl.kernel(
    _sc_body,
    out_type=(jax.ShapeDtypeStruct((N * K,), jnp.float32),
              jax.ShapeDtypeStruct((N * K,), jnp.float32)),
    mesh=plsc.VectorSubcoreMesh(core_axis_name="c", subcore_axis_name="s"),
    scratch_types=[
        pltpu.VMEM_SHARED((N,), jnp.float32),
        pltpu.VMEM_SHARED((N * K,), jnp.float32),
        pltpu.VMEM((CH2,), jnp.int32),
        pltpu.VMEM((CH2,), jnp.int32),
        pltpu.VMEM((CH1,), jnp.int32),
        pltpu.VMEM((CH1,), jnp.int32),
        pltpu.VMEM((CH2,), jnp.int32),
        pltpu.VMEM((CH2,), jnp.int32),
        pltpu.VMEM((CH2,), jnp.float32),
        pltpu.VMEM((CH2,), jnp.float32),
        pltpu.VMEM((CH2,), jnp.int32),
        pltpu.VMEM((CH2,), jnp.int32),
        pltpu.VMEM((CH2,), jnp.float32),
        pltpu.VMEM((ZB,), jnp.float32),
        pltpu.SemaphoreType.DMA,
        pltpu.SemaphoreType.DMA,
        pltpu.SemaphoreType.DMA,
        pltpu.SemaphoreType.DMA,
    ],
)


def _tc_body(c0_ref, c1_ref, bt_ref, w1_ref, b1_ref, w2_ref, b2_ref, ws_ref,
             bs_ref, out_ref):
    mm = lambda a, b: lax.dot_general(a, b, (((1,), (0,)), ((), ())))
    C = c0_ref[...] + c1_ref[...]
    c1 = jnp.sum(w1_ref[...], axis=0, keepdims=True)
    b1 = b1_ref[...]
    gp = jnp.where(c1 > 0, c1, 0.0)
    g0 = jnp.where(c1 > 0, b1, jnp.where(c1 == 0, jnp.maximum(b1, 0.0), 0.0))
    rowi = lax.broadcasted_iota(jnp.int32, (K, D), 0)
    drow = rowi.astype(jnp.float32)
    F = jnp.maximum(drow * c1 + b1, 0.0)
    F = jnp.where(rowi == OVD, g0, jnp.where(rowi == K - 1, gp, F))
    M = mm(F, w2_ref[...])
    h2 = jnp.maximum(mm(C, M) + b2_ref[...], 0.0)
    v = mm(h2, ws_ref[...])
    bt = bt_ref[...]
    oh = (lax.broadcasted_iota(jnp.int32, (G, N), 0) == bt).astype(jnp.float32)
    counts = jnp.sum(oh, axis=1, keepdims=True)
    sv = mm(oh, v)
    z = sv / jnp.maximum(counts, 1.0) + bs_ref[...]
    out_ref[...] = 1.0 / (1.0 + jnp.exp(-z))


_tc_dense = pl.pallas_call(
    _tc_body,
    out_shape=jax.ShapeDtypeStruct((G, 1), jnp.float32),
)


def kernel(x, edge_index, batch, W1, b1, W2, b2, Ws, bs):
    c0, c1 = _sc_build(edge_index.reshape(2 * E))
    return _tc_dense(c0.reshape(N, K), c1.reshape(N, K), batch.reshape(1, N),
                     W1, b1.reshape(1, D), W2, b2.reshape(1, D), Ws,
                     bs.reshape(1, 1))


# speedup vs baseline: 43.0414x; 1.0062x over previous
import jax
import jax.numpy as jnp
from jax import lax
from jax.experimental import pallas as pl
from jax.experimental.pallas import tpu as pltpu
from jax.experimental.pallas import tpu_sc as plsc

N = 10000
E = 320000
D = 128
G = 64
K = 128
OVD = K - 2

NC, NS = 2, 16
EPT1 = E // NS
EPT2 = E // (NC * NS)
CH1 = 2000
CH2 = 2000
ZB = 8000
CSLICE = (N * K) // NS


def _sc_body(e_hbm, out0_hbm, out1_hbm, deg_sh, c_sh,
             idx_a, idx_b, dst1_a, dst1_b, dst2_a, dst2_b,
             dval_a, dval_b, fidx_a, fidx_b, ones_v, zbuf_v,
             sg_a, sg_b, ss_a, ss_b, sz_a, sz_b):
    cid = lax.axis_index("c")
    sid = lax.axis_index("s")
    ones16 = jnp.ones((16,), jnp.float32)
    zeros16 = jnp.zeros((16,), jnp.float32)
    idxs = [idx_a, idx_b]
    dst1s = [dst1_a, dst1_b]
    dst2s = [dst2_a, dst2_b]
    dvals = [dval_a, dval_b]
    fidxs = [fidx_a, fidx_b]
    sgs = [sg_a, sg_b]
    sss = [ss_a, ss_b]

    def fill_zero(i, _):
        zbuf_v[pl.ds(i * 16, 16)] = zeros16
        return 0
    lax.fori_loop(0, ZB // 16, fill_zero, 0)

    def fill_one(i, _):
        ones_v[pl.ds(i * 16, 16)] = ones16
        return 0
    lax.fori_loop(0, CH2 // 16, fill_one, 0)

    hz = []
    for z in range(CSLICE // ZB):
        hz.append(pltpu.async_copy(
            zbuf_v, c_sh.at[pl.ds(sid * CSLICE + z * ZB, ZB)], sgs[z & 1]))

    @pl.when(sid == 0)
    def _():
        pltpu.sync_copy(zbuf_v, deg_sh.at[pl.ds(0, ZB)])
        pltpu.sync_copy(zbuf_v.at[pl.ds(0, N - ZB)],
                        deg_sh.at[pl.ds(ZB, N - ZB)])

    for h in hz:
        h.wait()
    plsc.subcore_barrier()

    dst1s4 = [dst1_a, dst1_b, idx_a, idx_b]
    ss4 = [ss_a, ss_b, sz_a, sz_b]
    hs = [None, None, None, None]
    for j in range(EPT1 // CH1):
        b = j & 3
        if hs[b] is not None:
            hs[b].wait()
        pltpu.sync_copy(e_hbm.at[pl.ds(E + sid * EPT1 + j * CH1, CH1)],
                        dst1s4[b])
        hs[b] = pltpu.async_copy(ones_v, deg_sh.at[dst1s4[b]], ss4[b],
                                 add=True)
    for h in hs:
        h.wait()

    plsc.subcore_barrier()

    base = cid * (E // NC) + sid * EPT2
    NJ = EPT2 // CH2
    pltpu.sync_copy(e_hbm.at[pl.ds(base, CH2)], idxs[0])
    pltpu.sync_copy(e_hbm.at[pl.ds(E + base, CH2)], dst2s[0])
    hg = [None, None]
    hg[0] = pltpu.async_copy(deg_sh.at[idxs[0]], dvals[0], sgs[0])
    hsc = [None, None]
    mxv = zeros16
    for j in range(NJ):
        b = j & 1
        if hsc[b] is not None:
            hsc[b].wait()
        hg[b].wait()
        if j + 1 < NJ:
            nb = (j + 1) & 1
            pltpu.sync_copy(e_hbm.at[pl.ds(base + (j + 1) * CH2, CH2)],
                            idxs[nb])
            pltpu.sync_copy(e_hbm.at[pl.ds(E + base + (j + 1) * CH2, CH2)],
                            dst2s[nb])
            hg[nb] = pltpu.async_copy(deg_sh.at[idxs[nb]], dvals[nb], sgs[nb])

        dval_c = dvals[b]
        dst_c = dst2s[b]
        fidx_c = fidxs[b]

        def vec(i, mx):
            d = dval_c[pl.ds(i * 16, 16)]
            di = d.astype(jnp.int32)
            dstw = dst_c[pl.ds(i * 16, 16)]
            dc = jnp.minimum(di, OVD)
            fidx_c[pl.ds(i * 16, 16)] = dstw * K + dc
            return jnp.maximum(mx, d)
        mxv = lax.fori_loop(0, CH2 // 16, vec, mxv)
        hsc[b] = pltpu.async_copy(ones_v, c_sh.at[fidx_c], sss[b], add=True)
    for h in hsc:
        if h is not None:
            h.wait()

    mx = mxv[0]
    for t in range(1, 16):
        mx = jnp.maximum(mx, mxv[t])

    @pl.when(mx >= float(OVD))
    def _():
        for j in range(NJ):
            pltpu.sync_copy(e_hbm.at[pl.ds(base + j * CH2, CH2)], idx_a)
            pltpu.sync_copy(e_hbm.at[pl.ds(E + base + j * CH2, CH2)], dst2_a)
            pltpu.sync_copy(deg_sh.at[idx_a], dval_a)

            def ovv(i, _):
                d = dval_a[pl.ds(i * 16, 16)]
                di = d.astype(jnp.int32)
                dstw = dst2_a[pl.ds(i * 16, 16)]
                fidx_a[pl.ds(i * 16, 16)] = dstw * K + (K - 1)
                dval_a[pl.ds(i * 16, 16)] = jnp.where(di >= OVD, d, 0.0)
                return 0
            lax.fori_loop(0, CH2 // 16, ovv, 0)
            pltpu.sync_copy(dval_a, c_sh.at[fidx_a], add=True)

    plsc.subcore_barrier()

    @pl.when(cid == 0)
    def _():
        pltpu.sync_copy(c_sh.at[pl.ds(sid * CSLICE, CSLICE)],
                        out0_hbm.at[pl.ds(sid * CSLICE, CSLICE)])

    @pl.when(cid == 1)
    def _():
        pltpu.sync_copy(c_sh.at[pl.ds(sid * CSLICE, CSLICE)],
                        out1_hbm.at[pl.ds(sid * CSLICE, CSLICE)])


_sc_build = pl.kernel(
    _sc_body,
    out_type=(jax.ShapeDtypeStruct((N * K,), jnp.float32),
              jax.ShapeDtypeStruct((N * K,), jnp.float32)),
    mesh=plsc.VectorSubcoreMesh(core_axis_name="c", subcore_axis_name="s"),
    scratch_types=[
        pltpu.VMEM_SHARED((N,), jnp.float32),
        pltpu.VMEM_SHARED((N * K,), jnp.float32),
        pltpu.VMEM((CH2,), jnp.int32),
        pltpu.VMEM((CH2,), jnp.int32),
        pltpu.VMEM((CH1,), jnp.int32),
        pltpu.VMEM((CH1,), jnp.int32),
        pltpu.VMEM((CH2,), jnp.int32),
        pltpu.VMEM((CH2,), jnp.int32),
        pltpu.VMEM((CH2,), jnp.float32),
        pltpu.VMEM((CH2,), jnp.float32),
        pltpu.VMEM((CH2,), jnp.int32),
        pltpu.VMEM((CH2,), jnp.int32),
        pltpu.VMEM((CH2,), jnp.float32),
        pltpu.VMEM((ZB,), jnp.float32),
        pltpu.SemaphoreType.DMA,
        pltpu.SemaphoreType.DMA,
        pltpu.SemaphoreType.DMA,
        pltpu.SemaphoreType.DMA,
        pltpu.SemaphoreType.DMA,
        pltpu.SemaphoreType.DMA,
    ],
)


def _tc_body(c0_ref, c1_ref, bt_ref, w1_ref, b1_ref, w2_ref, b2_ref, ws_ref,
             bs_ref, out_ref):
    mm = lambda a, b: lax.dot_general(a, b, (((1,), (0,)), ((), ())))
    C = c0_ref[...] + c1_ref[...]
    c1 = jnp.sum(w1_ref[...], axis=0, keepdims=True)
    b1 = b1_ref[...]
    gp = jnp.where(c1 > 0, c1, 0.0)
    g0 = jnp.where(c1 > 0, b1, jnp.where(c1 == 0, jnp.maximum(b1, 0.0), 0.0))
    rowi = lax.broadcasted_iota(jnp.int32, (K, D), 0)
    drow = rowi.astype(jnp.float32)
    F = jnp.maximum(drow * c1 + b1, 0.0)
    F = jnp.where(rowi == OVD, g0, jnp.where(rowi == K - 1, gp, F))
    M = mm(F, w2_ref[...])
    h2 = jnp.maximum(mm(C, M) + b2_ref[...], 0.0)
    v = mm(h2, ws_ref[...])
    bt = bt_ref[...]
    oh = (lax.broadcasted_iota(jnp.int32, (G, N), 0) == bt).astype(jnp.float32)
    counts = jnp.sum(oh, axis=1, keepdims=True)
    sv = mm(oh, v)
    z = sv / jnp.maximum(counts, 1.0) + bs_ref[...]
    out_ref[...] = 1.0 / (1.0 + jnp.exp(-z))


_tc_dense = pl.pallas_call(
    _tc_body,
    out_shape=jax.ShapeDtypeStruct((G, 1), jnp.float32),
)


def kernel(x, edge_index, batch, W1, b1, W2, b2, Ws, bs):
    c0, c1 = _sc_build(edge_index.reshape(2 * E))
    return _tc_dense(c0.reshape(N, K), c1.reshape(N, K), batch.reshape(1, N),
                     W1, b1.reshape(1, D), W2, b2.reshape(1, D), Ws,
                     bs.reshape(1, 1))
